# Initial kernel scaffold; baseline (speedup 1.0000x reference)
import jax
import jax.numpy as jnp
from jax.experimental import pallas as pl


def kernel(x, d):
    raise NotImplementedError("write your pallas kernel here")


# speedup vs baseline: 1.6505x; 1.6505x over previous
import dataclasses
import functools

import jax
import jax.numpy as jnp
from jax import lax
from jax.experimental import pallas as pl
from jax.experimental.pallas import tpu as pltpu
from jax.experimental.pallas import tpu_sc as plsc

N = 16384
C = 200
NC = 2
NS = 16
NW = NC * NS
L = 16
ROWS_PER_W = N // NW
CHUNK = 64
NCHUNKS = ROWS_PER_W // CHUNK
FLAT = CHUNK * C
NGROUPS = FLAT // L

_mesh = plsc.VectorSubcoreMesh(core_axis_name="c", subcore_axis_name="s")

_cp = pltpu.CompilerParams()
if "needs_layout_passes" in pltpu.CompilerParams.__dataclass_fields__:
  _cp = dataclasses.replace(_cp, needs_layout_passes=False)


@jax.jit
def _gather_sc(xf, df):
  @functools.partial(
      pl.kernel,
      out_type=jax.ShapeDtypeStruct((N * C,), jnp.float32),
      mesh=_mesh,
      scratch_types=[
          pltpu.VMEM((FLAT,), jnp.float32),
          pltpu.VMEM((FLAT,), jnp.int32),
          pltpu.VMEM((FLAT,), jnp.float32),
          pltpu.VMEM((FLAT,), jnp.int32),
          pltpu.SemaphoreType.DMA,
          pltpu.SemaphoreType.DMA,
          pltpu.SemaphoreType.DMA,
      ],
      compiler_params=_cp,
  )
  def k(x_hbm, d_hbm, o_hbm, xv, dv, ov, offv, semx, semd, semo):
    wid = lax.axis_index("s") * NC + lax.axis_index("c")
    base = wid * (ROWS_PER_W * C)
    lane = lax.iota(jnp.int32, L)

    @pl.loop(0, NGROUPS)
    def _(g):
      e = lane + g * L
      offv[pl.ds(g * L, L)] = (e // C) * C

    @pl.loop(0, NCHUNKS)
    def _(c):
      start = base + c * FLAT
      cpx = pltpu.async_copy(x_hbm.at[pl.ds(start, FLAT)], xv, semx)
      cpd = pltpu.async_copy(d_hbm.at[pl.ds(start, FLAT)], dv, semd)
      cpx.wait()
      cpd.wait()

      @pl.loop(0, NGROUPS)
      def _(g):
        sl = pl.ds(g * L, L)
        idx = dv[sl] + offv[sl]
        ov[sl] = plsc.load_gather(xv, [idx])

      pltpu.async_copy(ov, o_hbm.at[pl.ds(start, FLAT)], semo).wait()

  return k(xf, df)


def kernel(x, d):
  xf = x.reshape(-1)
  df = d.astype(jnp.int32).reshape(-1)
  return _gather_sc(xf, df).reshape(N, C)


# speedup vs baseline: 1.7296x; 1.0479x over previous
import dataclasses
import functools

import jax
import jax.numpy as jnp
from jax import lax
from jax.experimental import pallas as pl
from jax.experimental.pallas import tpu as pltpu
from jax.experimental.pallas import tpu_sc as plsc

N = 16384
C = 200
NC = 2
NS = 16
NW = NC * NS
L = 16
ROWS_PER_W = N // NW
CHUNK = 64
NCHUNKS = ROWS_PER_W // CHUNK
GROUP_OFFS = tuple(range(0, C - L + 1, L)) + (C - L,)

_mesh = plsc.VectorSubcoreMesh(core_axis_name="c", subcore_axis_name="s")

_cp = pltpu.CompilerParams()
if "needs_layout_passes" in pltpu.CompilerParams.__dataclass_fields__:
  _cp = dataclasses.replace(_cp, needs_layout_passes=False)


@jax.jit
def _gather_sc(x, d):
  @functools.partial(
      pl.kernel,
      out_type=jax.ShapeDtypeStruct((N, C), jnp.float32),
      mesh=_mesh,
      scratch_types=[
          pltpu.VMEM((CHUNK, C), jnp.float32),
          pltpu.VMEM((CHUNK, C), jnp.int32),
          pltpu.VMEM((CHUNK, C), jnp.float32),
          pltpu.SemaphoreType.DMA,
          pltpu.SemaphoreType.DMA,
          pltpu.SemaphoreType.DMA,
      ],
      compiler_params=_cp,
  )
  def k(x_hbm, d_hbm, o_hbm, xv, dv, ov, semx, semd, semo):
    wid = lax.axis_index("s") * NC + lax.axis_index("c")
    base = wid * ROWS_PER_W
    lane = lax.iota(jnp.int32, L)
    cols = [lane + o for o in GROUP_OFFS]

    @pl.loop(0, NCHUNKS)
    def _(c):
      r0 = base + c * CHUNK
      cpx = pltpu.async_copy(x_hbm.at[pl.ds(r0, CHUNK)], xv, semx)
      cpd = pltpu.async_copy(d_hbm.at[pl.ds(r0, CHUNK)], dv, semd)
      cpx.wait()
      cpd.wait()

      @pl.loop(0, CHUNK)
      def _(r):
        rsplat = jnp.zeros((L,), jnp.int32) + r
        for col in cols:
          idx = plsc.load_gather(dv, [rsplat, col])
          vals = plsc.load_gather(xv, [rsplat, idx])
          plsc.store_scatter(ov, [rsplat, col], vals)

      pltpu.async_copy(ov, o_hbm.at[pl.ds(r0, CHUNK)], semo).wait()

  return k(x, d)


def kernel(x, d):
  return _gather_sc(x, d.astype(jnp.int32))


# speedup vs baseline: 1.9206x; 1.1105x over previous
import dataclasses
import functools

import jax
import jax.numpy as jnp
from jax import lax
from jax.experimental import pallas as pl
from jax.experimental.pallas import tpu as pltpu
from jax.experimental.pallas import tpu_sc as plsc

N = 16384
C = 200
NC = 2
NS = 16
NW = NC * NS
L = 16
ROWS_PER_W = N // NW
CHUNK = 64
NCHUNKS = ROWS_PER_W // CHUNK
FLAT = CHUNK * C
NBUF = 2
GROUP_OFFS = tuple(range(0, C - L + 1, L)) + (C - L,)

_mesh = plsc.VectorSubcoreMesh(core_axis_name="c", subcore_axis_name="s")

_cp = pltpu.CompilerParams()
if "needs_layout_passes" in pltpu.CompilerParams.__dataclass_fields__:
  _cp = dataclasses.replace(_cp, needs_layout_passes=False)

_buf_types = []
for _ in range(NBUF):
  _buf_types += [
      pltpu.VMEM((CHUNK, C), jnp.float32),
      pltpu.VMEM((CHUNK, C), jnp.int32),
      pltpu.VMEM((CHUNK, C), jnp.float32),
      pltpu.SemaphoreType.DMA,
      pltpu.SemaphoreType.DMA,
      pltpu.SemaphoreType.DMA,
  ]


@jax.jit
def _gather_sc(x, d):
  @functools.partial(
      pl.kernel,
      out_type=jax.ShapeDtypeStruct((N, C), jnp.float32),
      mesh=_mesh,
      scratch_types=_buf_types,
      compiler_params=_cp,
  )
  def k(x_hbm, d_hbm, o_hbm, *bufs_flat):
    wid = lax.axis_index("s") * NC + lax.axis_index("c")
    base = wid * ROWS_PER_W
    lane = lax.iota(jnp.int32, L)
    bufs = [bufs_flat[6 * b:6 * (b + 1)] for b in range(NBUF)]

    pend_in = {}
    pend_out = {}

    def issue_in(cc):
      xv, dv, _, sx, sd, _ = bufs[cc % NBUF]
      r0 = base + cc * CHUNK
      pend_in[cc] = (
          pltpu.async_copy(x_hbm.at[pl.ds(r0, CHUNK)], xv, sx),
          pltpu.async_copy(d_hbm.at[pl.ds(r0, CHUNK)], dv, sd),
      )

    for cc in range(NBUF):
      issue_in(cc)

    cols = [lane + o for o in GROUP_OFFS]

    for cc in range(NCHUNKS):
      xv, dv, ov, _, _, so = bufs[cc % NBUF]
      cpx, cpd = pend_in.pop(cc)
      cpx.wait()
      cpd.wait()
      if cc - NBUF >= 0:
        pend_out.pop(cc - NBUF).wait()

      @pl.loop(0, CHUNK)
      def _(r):
        rsplat = jnp.zeros((L,), jnp.int32) + r
        for col in cols:
          idx = plsc.load_gather(dv, [rsplat, col])
          vals = plsc.load_gather(xv, [rsplat, idx])
          plsc.store_scatter(ov, [rsplat, col], vals)

      r0 = base + cc * CHUNK
      pend_out[cc] = pltpu.async_copy(ov, o_hbm.at[pl.ds(r0, CHUNK)], so)
      if cc + NBUF < NCHUNKS:
        issue_in(cc + NBUF)

    for cc in range(NCHUNKS - NBUF, NCHUNKS):
      pend_out.pop(cc).wait()

  return k(x, d)


def kernel(x, d):
  return _gather_sc(x, d.astype(jnp.int32))


# speedup vs baseline: 3.3197x; 1.7285x over previous
import dataclasses
import functools

import jax
import jax.numpy as jnp
from jax import lax
from jax.experimental import pallas as pl
from jax.experimental.pallas import tpu as pltpu
from jax.experimental.pallas import tpu_sc as plsc

N = 16384
C = 200
NC = 2
NS = 16
NW = NC * NS
L = 16
ROWS_PER_W = N // NW
CHUNK = 64
NCHUNKS = ROWS_PER_W // CHUNK
FLAT = CHUNK * C
NBUF = 2
GROUP_OFFS = tuple(range(0, C - L + 1, L)) + (C - L,)

_mesh = plsc.VectorSubcoreMesh(core_axis_name="c", subcore_axis_name="s")

_cp = pltpu.CompilerParams()
if "needs_layout_passes" in pltpu.CompilerParams.__dataclass_fields__:
  _cp = dataclasses.replace(_cp, needs_layout_passes=False)

_buf_types = []
for _ in range(NBUF):
  _buf_types += [
      pltpu.VMEM((CHUNK, C), jnp.float32),
      pltpu.VMEM((CHUNK, C), jnp.int32),
      pltpu.VMEM((CHUNK, C), jnp.float32),
      pltpu.SemaphoreType.DMA,
      pltpu.SemaphoreType.DMA,
      pltpu.SemaphoreType.DMA,
  ]


@jax.jit
def _gather_sc(x, d):
  @functools.partial(
      pl.kernel,
      out_type=jax.ShapeDtypeStruct((N, C), jnp.float32),
      mesh=_mesh,
      scratch_types=_buf_types,
      compiler_params=_cp,
  )
  def k(x_hbm, d_hbm, o_hbm, *bufs_flat):
    wid = lax.axis_index("s") * NC + lax.axis_index("c")
    base = wid * ROWS_PER_W
    lane = lax.iota(jnp.int32, L)
    bufs = [bufs_flat[6 * b:6 * (b + 1)] for b in range(NBUF)]

    pend_in = {}
    pend_out = {}

    def issue_in(cc):
      xv, dv, _, sx, sd, _ = bufs[cc % NBUF]
      r0 = base + cc * CHUNK
      pend_in[cc] = (
          pltpu.async_copy(x_hbm.at[pl.ds(r0, CHUNK)], xv, sx),
          pltpu.async_copy(d_hbm.at[pl.ds(r0, CHUNK)], dv, sd),
      )

    for cc in range(NBUF):
      issue_in(cc)

    cols = [lane + o for o in GROUP_OFFS]

    for cc in range(NCHUNKS):
      xv, dv, ov, _, _, so = bufs[cc % NBUF]
      cpx, cpd = pend_in.pop(cc)
      cpx.wait()
      cpd.wait()
      if cc - NBUF >= 0:
        pend_out.pop(cc - NBUF).wait()

      @plsc.parallel_loop(0, CHUNK, unroll=2)
      def _(r):
        rsplat = jnp.zeros((L,), jnp.int32) + r
        for col in cols:
          idx = plsc.load_gather(dv, [rsplat, col])
          vals = plsc.load_gather(xv, [rsplat, idx])
          plsc.store_scatter(ov, [rsplat, col], vals)

      r0 = base + cc * CHUNK
      pend_out[cc] = pltpu.async_copy(ov, o_hbm.at[pl.ds(r0, CHUNK)], so)
      if cc + NBUF < NCHUNKS:
        issue_in(cc + NBUF)

    for cc in range(NCHUNKS - NBUF, NCHUNKS):
      pend_out.pop(cc).wait()

  return k(x, d)


def kernel(x, d):
  return _gather_sc(x, d.astype(jnp.int32))
